# baseline (speedup 1.0000x reference)
import functools

import jax
import jax.numpy as jnp
from jax import lax
from jax.experimental import pallas as pl
from jax.experimental.pallas import tpu as pltpu
from jax.experimental.pallas import tpu_sc as plsc

_MAX_LEN = 8192
_D_MODEL = 768
_BATCH = 4
_BATCH_SC = 1
_BATCH_TC = _BATCH - _BATCH_SC
_NUM_CORES = 2
_NUM_SUBCORES = 16
_NUM_WORKERS = _NUM_CORES * _NUM_SUBCORES
_ROWS_PER_WORKER = _MAX_LEN // _NUM_WORKERS
_CHUNK_ROWS = 64
_NUM_CHUNKS = _ROWS_PER_WORKER // _CHUNK_ROWS
_BLK = 256


def _make_sc_broadcast():
  mesh = plsc.VectorSubcoreMesh(core_axis_name="c", subcore_axis_name="s")

  @functools.partial(
      pl.kernel,
      mesh=mesh,
      out_type=jax.ShapeDtypeStruct((_BATCH_SC, _MAX_LEN, _D_MODEL),
                                    jnp.float32),
      scratch_types=[
          pltpu.VMEM((_CHUNK_ROWS, _D_MODEL), jnp.float32),
          pltpu.VMEM((_CHUNK_ROWS, _D_MODEL), jnp.float32),
          pltpu.SemaphoreType.DMA,
          pltpu.SemaphoreType.DMA,
          pltpu.SemaphoreType.DMA,
          pltpu.SemaphoreType.DMA,
      ],
  )
  def broadcast_kernel(table_hbm, out_hbm, buf0, buf1, gsem0, gsem1,
                       ssem0, ssem1):
    wid = lax.axis_index("s") * _NUM_CORES + lax.axis_index("c")
    base = wid * _ROWS_PER_WORKER
    bufs = (buf0, buf1)
    gsems = (gsem0, gsem1)
    ssems = (ssem0, ssem1)

    def rows(i):
      return pl.ds(base + i * _CHUNK_ROWS, _CHUNK_ROWS)

    gathers = [None] * _NUM_CHUNKS
    pending_scatters = [[], []]
    gathers[0] = pltpu.async_copy(table_hbm.at[rows(0)], bufs[0], gsems[0])
    for i in range(_NUM_CHUNKS):
      bi = i % 2
      if i + 1 < _NUM_CHUNKS:
        ni = (i + 1) % 2
        for c in pending_scatters[ni]:
          c.wait()
        pending_scatters[ni] = []
        gathers[i + 1] = pltpu.async_copy(table_hbm.at[rows(i + 1)],
                                          bufs[ni], gsems[ni])
      gathers[i].wait()
      pending_scatters[bi] = [
          pltpu.async_copy(bufs[bi], out_hbm.at[b, rows(i)], ssems[bi])
          for b in range(_BATCH_SC)
      ]
    for lst in pending_scatters:
      for c in lst:
        c.wait()

  return broadcast_kernel


_sc_broadcast = _make_sc_broadcast()


def _tc_body(t_ref, o_ref):
  o_ref[...] = jnp.broadcast_to(t_ref[...][None], (_BATCH_TC, _BLK, _D_MODEL))


_tc_broadcast = pl.pallas_call(
    _tc_body,
    grid=(_MAX_LEN // _BLK,),
    in_specs=[pl.BlockSpec((_BLK, _D_MODEL), lambda i: (i, 0))],
    out_specs=pl.BlockSpec((_BATCH_TC, _BLK, _D_MODEL), lambda i: (0, i, 0)),
    out_shape=jax.ShapeDtypeStruct((_BATCH_TC, _MAX_LEN, _D_MODEL),
                                   jnp.float32),
)


@jax.jit
def kernel(x, pe_table):
  del x
  sc_part = _sc_broadcast(pe_table)
  tc_part = _tc_broadcast(pe_table)
  return jnp.concatenate([sc_part, tc_part], axis=0)


# speedup vs baseline: 2.1302x; 2.1302x over previous
import functools

import jax
import jax.numpy as jnp
from jax import lax
from jax.experimental import pallas as pl
from jax.experimental.pallas import tpu as pltpu
from jax.experimental.pallas import tpu_sc as plsc

_MAX_LEN = 8192
_D_MODEL = 768
_BATCH = 4
_NUM_CORES = 2
_NUM_SUBCORES = 16
_NUM_WORKERS = _NUM_CORES * _NUM_SUBCORES
_ROWS_PER_WORKER = _MAX_LEN // _NUM_WORKERS
_CHUNK_ROWS = 32
_NBUF = 4
_NUM_CHUNKS = _ROWS_PER_WORKER // _CHUNK_ROWS


def _make_sc_broadcast():
  mesh = plsc.VectorSubcoreMesh(core_axis_name="c", subcore_axis_name="s")

  scratch = ([pltpu.VMEM((_CHUNK_ROWS, _D_MODEL), jnp.float32)] * _NBUF
             + [pltpu.SemaphoreType.DMA] * (2 * _NBUF))

  @functools.partial(
      pl.kernel,
      mesh=mesh,
      out_type=jax.ShapeDtypeStruct((_BATCH, _MAX_LEN, _D_MODEL),
                                    jnp.float32),
      scratch_types=scratch,
  )
  def broadcast_kernel(table_hbm, out_hbm, *scratch_refs):
    bufs = scratch_refs[:_NBUF]
    gsems = scratch_refs[_NBUF:2 * _NBUF]
    ssems = scratch_refs[2 * _NBUF:]
    wid = lax.axis_index("s") * _NUM_CORES + lax.axis_index("c")
    base = wid * _ROWS_PER_WORKER

    def rows(i):
      return pl.ds(base + i * _CHUNK_ROWS, _CHUNK_ROWS)

    gathers = [None] * _NUM_CHUNKS
    pending_scatters = [[] for _ in range(_NBUF)]
    for j in range(min(_NBUF, _NUM_CHUNKS)):
      gathers[j] = pltpu.async_copy(table_hbm.at[rows(j)], bufs[j],
                                    gsems[j])
    for i in range(_NUM_CHUNKS):
      bi = i % _NBUF
      gathers[i].wait()
      pending_scatters[bi] = [
          pltpu.async_copy(bufs[bi], out_hbm.at[b, rows(i)], ssems[bi])
          for b in range(_BATCH)
      ]
      k = i + 1
      if _NBUF <= k < _NUM_CHUNKS:
        ki = k % _NBUF
        for c in pending_scatters[ki]:
          c.wait()
        pending_scatters[ki] = []
        gathers[k] = pltpu.async_copy(table_hbm.at[rows(k)], bufs[ki],
                                      gsems[ki])
    for lst in pending_scatters:
      for c in lst:
        c.wait()

  return broadcast_kernel


_sc_broadcast = _make_sc_broadcast()


@jax.jit
def kernel(x, pe_table):
  del x
  return _sc_broadcast(pe_table)


# speedup vs baseline: 2.1709x; 1.0191x over previous
import functools

import jax
import jax.numpy as jnp
from jax import lax
from jax.experimental import pallas as pl
from jax.experimental.pallas import tpu as pltpu
from jax.experimental.pallas import tpu_sc as plsc

_MAX_LEN = 8192
_D_MODEL = 768
_BATCH = 4
_NUM_CORES = 2
_NUM_SUBCORES = 16
_NUM_WORKERS = _NUM_CORES * _NUM_SUBCORES
_ROWS_PER_WORKER = _MAX_LEN // _NUM_WORKERS
_CHUNK_ROWS = 64
_NUM_CHUNKS = _ROWS_PER_WORKER // _CHUNK_ROWS


def _make_sc_broadcast():
  mesh = plsc.VectorSubcoreMesh(core_axis_name="c", subcore_axis_name="s")

  @functools.partial(
      pl.kernel,
      mesh=mesh,
      out_type=jax.ShapeDtypeStruct((_BATCH, _MAX_LEN, _D_MODEL),
                                    jnp.float32),
      scratch_types=[
          pltpu.VMEM((_CHUNK_ROWS, _D_MODEL), jnp.float32),
          pltpu.VMEM((_CHUNK_ROWS, _D_MODEL), jnp.float32),
          pltpu.SemaphoreType.DMA,
          pltpu.SemaphoreType.DMA,
          pltpu.SemaphoreType.DMA,
          pltpu.SemaphoreType.DMA,
      ],
  )
  def broadcast_kernel(table_hbm, out_hbm, buf0, buf1, gsem0, gsem1,
                       ssem0, ssem1):
    wid = lax.axis_index("s") * _NUM_CORES + lax.axis_index("c")
    base = wid * _ROWS_PER_WORKER
    bufs = (buf0, buf1)
    gsems = (gsem0, gsem1)
    ssems = (ssem0, ssem1)

    def rows(i):
      return pl.ds(base + i * _CHUNK_ROWS, _CHUNK_ROWS)

    gathers = [None] * (_NUM_CHUNKS + 1)
    pending_scatters = [[], []]
    gathers[0] = pltpu.async_copy(table_hbm.at[rows(0)], bufs[0], gsems[0])
    for i in range(_NUM_CHUNKS):
      bi = i % 2
      ni = (i + 1) % 2
      gathers[i].wait()
      new_scatters = [
          pltpu.async_copy(bufs[bi], out_hbm.at[b, rows(i)], ssems[bi])
          for b in range(_BATCH)
      ]
      if i + 1 < _NUM_CHUNKS:
        for c in pending_scatters[ni]:
          c.wait()
        pending_scatters[ni] = []
        gathers[i + 1] = pltpu.async_copy(table_hbm.at[rows(i + 1)],
                                          bufs[ni], gsems[ni])
      pending_scatters[bi] = new_scatters
    for lst in pending_scatters:
      for c in lst:
        c.wait()

  return broadcast_kernel


_sc_broadcast = _make_sc_broadcast()


@jax.jit
def kernel(x, pe_table):
  del x
  return _sc_broadcast(pe_table)


# speedup vs baseline: 2.2009x; 1.0138x over previous
import functools

import jax
import jax.numpy as jnp
from jax import lax
from jax.experimental import pallas as pl
from jax.experimental.pallas import tpu as pltpu
from jax.experimental.pallas import tpu_sc as plsc

_MAX_LEN = 8192
_D_MODEL = 768
_BATCH = 4
_NUM_CORES = 2
_NUM_SUBCORES = 16
_NUM_WORKERS = _NUM_CORES * _NUM_SUBCORES
_ROWS_PER_WORKER = _MAX_LEN // _NUM_WORKERS
_CHUNK_ROWS = 64
_NUM_CHUNKS = _ROWS_PER_WORKER // _CHUNK_ROWS


def _make_sc_broadcast():
  mesh = plsc.VectorSubcoreMesh(core_axis_name="c", subcore_axis_name="s")

  @functools.partial(
      pl.kernel,
      mesh=mesh,
      out_type=jax.ShapeDtypeStruct((_BATCH, _MAX_LEN, _D_MODEL),
                                    jnp.float32),
      scratch_types=[
          pltpu.VMEM((_CHUNK_ROWS, _D_MODEL), jnp.float32),
          pltpu.VMEM((_CHUNK_ROWS, _D_MODEL), jnp.float32),
          pltpu.SemaphoreType.DMA,
          pltpu.SemaphoreType.DMA,
          pltpu.SemaphoreType.DMA,
          pltpu.SemaphoreType.DMA,
      ],
  )
  def broadcast_kernel(table_hbm, out_hbm, buf0, buf1, gsem0, gsem1,
                       ssem0, ssem1):
    wid = lax.axis_index("s") * _NUM_CORES + lax.axis_index("c")
    base = wid * _ROWS_PER_WORKER
    bufs = (buf0, buf1)
    gsems = (gsem0, gsem1)
    ssems = (ssem0, ssem1)

    def rows(i):
      return pl.ds(base + i * _CHUNK_ROWS, _CHUNK_ROWS)

    gathers = [None] * _NUM_CHUNKS
    pending_scatters = [[], []]
    gathers[0] = pltpu.async_copy(table_hbm.at[rows(0)], bufs[0], gsems[0])
    for i in range(_NUM_CHUNKS):
      bi = i % 2
      if i + 1 < _NUM_CHUNKS:
        ni = (i + 1) % 2
        for c in pending_scatters[ni]:
          c.wait()
        pending_scatters[ni] = []
        gathers[i + 1] = pltpu.async_copy(table_hbm.at[rows(i + 1)],
                                          bufs[ni], gsems[ni])
      gathers[i].wait()
      pending_scatters[bi] = [
          pltpu.async_copy(bufs[bi], out_hbm.at[b, rows(i)], ssems[bi])
          for b in range(_BATCH)
      ]
    for lst in pending_scatters:
      for c in lst:
        c.wait()

  return broadcast_kernel


_sc_broadcast = _make_sc_broadcast()


@jax.jit
def kernel(x, pe_table):
  del x
  return _sc_broadcast(pe_table)
